# baseline (speedup 1.0000x reference)
import functools

import jax
import jax.numpy as jnp
from jax import lax
from jax.experimental import pallas as pl
from jax.experimental.pallas import tpu as pltpu
from jax.experimental.pallas import tpu_sc as plsc

_L = 16
_D = 256


def _compute_rows(y_ref, c_ref, w_ref, y_v, c_v, rows_v):
    pltpu.sync_copy(y_ref, y_v)
    pltpu.sync_copy(c_ref, c_v)
    cv = c_v[pl.ds(16, _L)]
    c17 = jnp.full((_L,), cv[1], jnp.float32)
    c18 = jnp.full((_L,), cv[2], jnp.float32)
    c19 = jnp.full((_L,), cv[3], jnp.float32)
    row0 = c19 * c17
    row1 = c18 / c19
    for j in range(_D // _L):
        o = j * _L
        rows_v[pl.ds(o, _L)] = row0
        rows_v[pl.ds(_D + o, _L)] = row1
        rows_v[pl.ds(2 * _D + o, _L)] = (
            y_v[pl.ds(2 * _D + o, _L)]
            + y_v[pl.ds(o, _L)]
            + 2.0 * y_v[pl.ds(_D + o, _L)]
        )
    pltpu.sync_copy(rows_v, w_ref.at[pl.ds(0, 3 * _D)])


def _update_body(y_ref, c_ref, w_ref, y_v, c_v, rows_v):
    cid = lax.axis_index("c")
    sid = lax.axis_index("s")

    @pl.when(jnp.logical_and(cid == 0, sid == 0))
    def _():
        _compute_rows(y_ref, c_ref, w_ref, y_v, c_v, rows_v)


@functools.lru_cache(maxsize=None)
def _make_update():
    return pl.kernel(
        _update_body,
        out_type=(),
        mesh=plsc.VectorSubcoreMesh(
            core_axis_name="c", subcore_axis_name="s",
            num_cores=2, num_subcores=16,
        ),
        scratch_types=[
            pltpu.VMEM((3 * _D,), jnp.float32),
            pltpu.VMEM((32,), jnp.float32),
            pltpu.VMEM((3 * _D,), jnp.float32),
        ],
    )


_N = 65536 * 256
_CHUNKS = 8


def _copy_body(w_ref, out_ref, *sems):
    chunk = _N // _CHUNKS
    copies = []
    for k in range(_CHUNKS):
        sl = pl.ds(k * chunk, chunk)
        cp = pltpu.make_async_copy(w_ref.at[sl], out_ref.at[sl], sems[k])
        cp.start()
        copies.append(cp)
    for cp in copies:
        cp.wait()


def _tc_copy(w_flat):
    return pl.pallas_call(
        _copy_body,
        out_shape=jax.ShapeDtypeStruct((_N,), jnp.float32),
        in_specs=[pl.BlockSpec(memory_space=pl.ANY)],
        out_specs=pl.BlockSpec(memory_space=pl.ANY),
        scratch_shapes=[pltpu.SemaphoreType.DMA] * _CHUNKS,
    )(w_flat)


def kernel(y, w, c, t):
    del t
    y_flat = y[1:4].reshape(-1)
    c_pad = jnp.pad(c, (0, 11))
    bulk = _tc_copy(w.reshape(-1))
    w_ref = jax.new_ref(bulk)
    _make_update()(y_flat, c_pad, w_ref)
    return jax.freeze(w_ref).reshape(w.shape)


# speedup vs baseline: 50.6343x; 50.6343x over previous
import functools

import jax
import jax.numpy as jnp
from jax import lax
from jax.experimental import pallas as pl
from jax.experimental.pallas import tpu as pltpu
from jax.experimental.pallas import tpu_sc as plsc

_ROWS = 65536
_D = 256
_BLK = 4096


def _fused_body(y_ref, c_ref, w_ref, out_ref):
    out_ref[...] = w_ref[...]

    @pl.when(pl.program_id(0) == 0)
    def _():
        c17 = c_ref[17]
        c18 = c_ref[18]
        c19 = c_ref[19]
        out_ref[0:1, :] = jnp.full((1, _D), c19 * c17, jnp.float32)
        out_ref[1:2, :] = jnp.full((1, _D), c18 / c19, jnp.float32)
        out_ref[2:3, :] = y_ref[3:4, :] + y_ref[1:2, :] + 2.0 * y_ref[2:3, :]


def _fused(y, w, c):
    grid = (_ROWS // _BLK,)
    return pl.pallas_call(
        _fused_body,
        out_shape=jax.ShapeDtypeStruct((_ROWS, _D), jnp.float32),
        grid=grid,
        in_specs=[
            pl.BlockSpec((8, _D), lambda i: (0, 0)),
            pl.BlockSpec(memory_space=pltpu.SMEM),
            pl.BlockSpec((_BLK, _D), lambda i: (i, 0)),
        ],
        out_specs=pl.BlockSpec((_BLK, _D), lambda i: (i, 0)),
        compiler_params=pltpu.CompilerParams(
            dimension_semantics=("arbitrary",),
        ),
    )(y, c, w)


def kernel(y, w, c, t):
    del t
    return _fused(y, w, c)


# speedup vs baseline: 52.3806x; 1.0345x over previous
import functools

import jax
import jax.numpy as jnp
from jax import lax
from jax.experimental import pallas as pl
from jax.experimental.pallas import tpu as pltpu
from jax.experimental.pallas import tpu_sc as plsc

_ROWS = 65536
_D = 256
_BLK = 8192


def _fused_body(y_ref, c_ref, w_ref, out_ref):
    out_ref[...] = w_ref[...]

    @pl.when(pl.program_id(0) == 0)
    def _():
        c17 = c_ref[17]
        c18 = c_ref[18]
        c19 = c_ref[19]
        out_ref[0:1, :] = jnp.full((1, _D), c19 * c17, jnp.float32)
        out_ref[1:2, :] = jnp.full((1, _D), c18 / c19, jnp.float32)
        out_ref[2:3, :] = y_ref[3:4, :] + y_ref[1:2, :] + 2.0 * y_ref[2:3, :]


def _fused(y, w, c):
    grid = (_ROWS // _BLK,)
    return pl.pallas_call(
        _fused_body,
        out_shape=jax.ShapeDtypeStruct((_ROWS, _D), jnp.float32),
        grid=grid,
        in_specs=[
            pl.BlockSpec((8, _D), lambda i: (0, 0)),
            pl.BlockSpec(memory_space=pltpu.SMEM),
            pl.BlockSpec((_BLK, _D), lambda i: (i, 0)),
        ],
        out_specs=pl.BlockSpec((_BLK, _D), lambda i: (i, 0)),
        compiler_params=pltpu.CompilerParams(
            dimension_semantics=("arbitrary",),
        ),
    )(y, c, w)


def kernel(y, w, c, t):
    del t
    return _fused(y, w, c)
